# baseline (speedup 1.0000x reference)
import functools

import jax
import jax.numpy as jnp
from jax import lax
from jax.experimental import pallas as pl
from jax.experimental.pallas import tpu as pltpu
from jax.experimental.pallas import tpu_sc as plsc

E_LAYERS = 6
NUM_EXPERTS = 8
NUM_HEADS = 12
HEAD_DIM = 64
B = 128
D = NUM_HEADS * NUM_EXPERTS * HEAD_DIM
NC = 2
NS = 16
NW = NC * NS
ROWS_PER_W = (2 * B) // NW


_mesh = plsc.VectorSubcoreMesh(core_axis_name="c", subcore_axis_name="s")


@functools.partial(
    pl.kernel,
    mesh=_mesh,
    out_type=[
        jax.ShapeDtypeStruct((B, D), jnp.float32),
        jax.ShapeDtypeStruct((B, D), jnp.float32),
    ],
    scratch_types=[
        pltpu.VMEM((1,), jnp.int32),
        pltpu.VMEM((1, D), jnp.float32),
        pltpu.SemaphoreType.DMA,
    ],
)
def _gather_broadcast(pk_hbm, pv_hbm, idx_hbm, ek_hbm, ev_hbm,
                      idx_v, row_v, sem):
    wid = lax.axis_index("s") * NC + lax.axis_index("c")
    pltpu.sync_copy(idx_hbm, idx_v)

    def _bcast_rows(table_hbm, out_hbm, base):
        pltpu.async_copy(table_hbm.at[idx_v], row_v, sem).wait()
        copies = [
            pltpu.async_copy(row_v, out_hbm.at[pl.ds(base + r, 1)], sem)
            for r in range(ROWS_PER_W)
        ]
        for c in copies:
            c.wait()

    @pl.when(wid < NS)
    def _ek():
        _bcast_rows(pk_hbm, ek_hbm, wid * ROWS_PER_W)

    @pl.when(wid >= NS)
    def _ev():
        _bcast_rows(pv_hbm, ev_hbm, (wid - NS) * ROWS_PER_W)


def kernel(x_querry, l, x_block, e_pk, e_pv):
    pk2 = e_pk.reshape(E_LAYERS, D)
    pv2 = e_pv.reshape(E_LAYERS, D)
    idx = jnp.asarray(l, dtype=jnp.int32).reshape(1)
    ek2, ev2 = _gather_broadcast(pk2, pv2, idx)
    xb = x_block * jnp.float32(1.0)
    Ek = ek2.reshape(B, NUM_HEADS, NUM_EXPERTS, HEAD_DIM)
    Ev = ev2.reshape(B, NUM_HEADS, NUM_EXPERTS, HEAD_DIM)
    eps_decay = jnp.full((NUM_HEADS, NUM_EXPERTS), 2.0, dtype=jnp.float32)
    loss = jnp.float32(0.0)
    return (Ek, Ev, eps_decay, loss, xb)


# speedup vs baseline: 1.0333x; 1.0333x over previous
import functools

import jax
import jax.numpy as jnp
from jax import lax
from jax.experimental import pallas as pl
from jax.experimental.pallas import tpu as pltpu
from jax.experimental.pallas import tpu_sc as plsc

E_LAYERS = 6
NUM_EXPERTS = 8
NUM_HEADS = 12
HEAD_DIM = 64
B = 128
D = NUM_HEADS * NUM_EXPERTS * HEAD_DIM
NC = 2
NS = 16
NW = NC * NS
ROWS_PER_W = (2 * B) // NW


_mesh = plsc.VectorSubcoreMesh(core_axis_name="c", subcore_axis_name="s")


@functools.partial(
    pl.kernel,
    mesh=_mesh,
    out_type=[
        jax.ShapeDtypeStruct((B, D), jnp.float32),
        jax.ShapeDtypeStruct((B, D), jnp.float32),
    ],
    scratch_types=[
        pltpu.VMEM((1,), jnp.int32),
        pltpu.VMEM((1, D), jnp.float32),
        pltpu.SemaphoreType.DMA,
    ],
)
def _gather_broadcast(pk_hbm, pv_hbm, idx_hbm, ek_hbm, ev_hbm,
                      idx_v, row_v, sem):
    wid = lax.axis_index("s") * NC + lax.axis_index("c")
    pltpu.sync_copy(idx_hbm, idx_v)

    def _bcast_rows(table_hbm, out_hbm, base):
        pltpu.async_copy(table_hbm.at[idx_v], row_v, sem).wait()
        copies = [
            pltpu.async_copy(row_v, out_hbm.at[pl.ds(base + r, 1)], sem)
            for r in range(ROWS_PER_W)
        ]
        for c in copies:
            c.wait()

    @pl.when(wid < NS)
    def _ek():
        _bcast_rows(pk_hbm, ek_hbm, wid * ROWS_PER_W)

    @pl.when(wid >= NS)
    def _ev():
        _bcast_rows(pv_hbm, ev_hbm, (wid - NS) * ROWS_PER_W)


def kernel(x_querry, l, x_block, e_pk, e_pv):
    pk2 = e_pk.reshape(E_LAYERS, D)
    pv2 = e_pv.reshape(E_LAYERS, D)
    idx = jnp.asarray(l, dtype=jnp.int32).reshape(1)
    ek2, ev2 = _gather_broadcast(pk2, pv2, idx)
    one = x_querry[0, 0] * jnp.float32(0.0) + jnp.float32(1.0)
    xb = x_block * one
    Ek = ek2.reshape(B, NUM_HEADS, NUM_EXPERTS, HEAD_DIM)
    Ev = ev2.reshape(B, NUM_HEADS, NUM_EXPERTS, HEAD_DIM)
    eps_decay = jnp.full((NUM_HEADS, NUM_EXPERTS), 2.0, dtype=jnp.float32)
    loss = jnp.float32(0.0)
    return (Ek, Ev, eps_decay, loss, xb)


# speedup vs baseline: 1.0371x; 1.0036x over previous
import functools

import jax
import jax.numpy as jnp
from jax import lax
from jax.experimental import pallas as pl
from jax.experimental.pallas import tpu as pltpu
from jax.experimental.pallas import tpu_sc as plsc

E_LAYERS = 6
NUM_EXPERTS = 8
NUM_HEADS = 12
HEAD_DIM = 64
B = 128
D = NUM_HEADS * NUM_EXPERTS * HEAD_DIM
NC = 2
NS = 16
NW = NC * NS
ROWS_PER_W = (2 * B) // NW


_mesh = plsc.VectorSubcoreMesh(core_axis_name="c", subcore_axis_name="s")


@functools.partial(
    pl.kernel,
    mesh=_mesh,
    out_type=[
        jax.ShapeDtypeStruct((B, D), jnp.float32),
        jax.ShapeDtypeStruct((B, D), jnp.float32),
    ],
    scratch_types=[
        pltpu.VMEM((1,), jnp.int32),
        pltpu.VMEM((1, D), jnp.float32),
        pltpu.SemaphoreType.DMA,
    ],
    compiler_params=pltpu.CompilerParams(skip_device_barrier=True),
)
def _gather_broadcast(pk_hbm, pv_hbm, idx_hbm, ek_hbm, ev_hbm,
                      idx_v, row_v, sem):
    wid = lax.axis_index("s") * NC + lax.axis_index("c")
    pltpu.sync_copy(idx_hbm, idx_v)

    def _bcast_rows(table_hbm, out_hbm, base):
        pltpu.async_copy(table_hbm.at[idx_v], row_v, sem).wait()
        copies = [
            pltpu.async_copy(row_v, out_hbm.at[pl.ds(base + r, 1)], sem)
            for r in range(ROWS_PER_W)
        ]
        for c in copies:
            c.wait()

    @pl.when(wid < NS)
    def _ek():
        _bcast_rows(pk_hbm, ek_hbm, wid * ROWS_PER_W)

    @pl.when(wid >= NS)
    def _ev():
        _bcast_rows(pv_hbm, ev_hbm, (wid - NS) * ROWS_PER_W)


def kernel(x_querry, l, x_block, e_pk, e_pv):
    pk2 = e_pk.reshape(E_LAYERS, D)
    pv2 = e_pv.reshape(E_LAYERS, D)
    idx = jnp.asarray(l, dtype=jnp.int32).reshape(1)
    ek2, ev2 = _gather_broadcast(pk2, pv2, idx)
    one = x_querry[0, 0] * jnp.float32(0.0) + jnp.float32(1.0)
    xb = x_block * one
    Ek = ek2.reshape(B, NUM_HEADS, NUM_EXPERTS, HEAD_DIM)
    Ev = ev2.reshape(B, NUM_HEADS, NUM_EXPERTS, HEAD_DIM)
    eps_decay = jnp.full((NUM_HEADS, NUM_EXPERTS), 2.0, dtype=jnp.float32)
    loss = jnp.float32(0.0)
    return (Ek, Ev, eps_decay, loss, xb)
